# Initial kernel scaffold; baseline (speedup 1.0000x reference)
import jax
import jax.numpy as jnp
from jax.experimental import pallas as pl


def kernel(xyz, cls_label, params):
    raise NotImplementedError("write your pallas kernel here")


# speedup vs baseline: 4.4820x; 4.4820x over previous
import functools
import math

import jax
import jax.numpy as jnp
import numpy as np
from jax import lax
from jax.experimental import pallas as pl
from jax.experimental.pallas import tpu as pltpu
from jax.experimental.pallas import tpu_sc as plsc

_BN_EPS = 1e-5


def _fold(layers):
    out = []
    for (W, b, g, be) in layers:
        s = g / np.sqrt(1.0 + _BN_EPS)
        out.append((W * s[None, :], b * s + be))
    return out


def _fps_body(P, N, xyz_ref, out_ref):
    x = xyz_ref[0]
    y = xyz_ref[1]
    z = xyz_ref[2]
    B = x.shape[0]
    iota = lax.broadcasted_iota(jnp.int32, (B, N), 1)
    iotaP = lax.broadcasted_iota(jnp.int32, (B, P), 1)

    def body(i, state):
        dist, far, cxs, cys, czs = state
        oh = iota == far
        cx = jnp.sum(jnp.where(oh, x, 0.0), axis=1, keepdims=True)
        cy = jnp.sum(jnp.where(oh, y, 0.0), axis=1, keepdims=True)
        cz = jnp.sum(jnp.where(oh, z, 0.0), axis=1, keepdims=True)
        sel = iotaP == i
        cxs = jnp.where(sel, cx, cxs)
        cys = jnp.where(sel, cy, cys)
        czs = jnp.where(sel, cz, czs)
        dx = x - cx
        dy = y - cy
        dz = z - cz
        d = (dx * dx + dy * dy) + dz * dz
        dist = jnp.minimum(dist, d)
        m = jnp.max(dist, axis=1, keepdims=True)
        far = jnp.min(jnp.where(dist == m, iota, N), axis=1, keepdims=True)
        return dist, far, cxs, cys, czs

    zP = jnp.zeros((B, P), jnp.float32)
    _, _, cxs, cys, czs = lax.fori_loop(
        0, P,
        body,
        (jnp.full((B, N), 1e10, jnp.float32), jnp.zeros((B, 1), jnp.int32),
         zP, zP, zP),
    )
    out_ref[0] = cxs
    out_ref[1] = cys
    out_ref[2] = czs


def _fps(xyz_sep, P):
    three, B, N = xyz_sep.shape
    return pl.pallas_call(
        functools.partial(_fps_body, P, N),
        out_shape=jax.ShapeDtypeStruct((3, B, P), jnp.float32),
    )(xyz_sep)


def _ballq_body(N, K, r2, xyz_ref, new_ref, slot_ref, cnt_ref):
    xyz = xyz_ref[...].reshape(3, N)
    x = xyz[0:1]
    y = xyz[1:2]
    z = xyz[2:3]
    nw = new_ref[...]
    Sc = nw.shape[1]
    nw = nw.reshape(Sc, 3)
    cx = nw[:, 0:1]
    cy = nw[:, 1:2]
    cz = nw[:, 2:3]
    x2n = (x * x + y * y) + z * z
    c2 = (cx * cx + cy * cy) + cz * cz
    cross = (cx * x + cy * y) + cz * z
    sqr = (c2 + x2n) - 2.0 * cross
    mask = jnp.where(sqr <= r2, 1.0, 0.0).astype(jnp.float32)
    run = mask
    sh = 1
    while sh < N:
        shifted = jnp.concatenate(
            [jnp.zeros((Sc, sh), jnp.float32), run[:, : N - sh]], axis=1)
        run = run + shifted
        sh *= 2
    excl = run - mask
    sloti = excl.astype(jnp.int32)
    slot_ref[...] = jnp.where(
        (mask > 0.0) & (sloti < K), sloti, -1).reshape(1, Sc, N)
    cnt = jnp.minimum(run[:, N - 1 : N], float(K)).astype(jnp.int32)
    cnt_ref[...] = cnt.reshape(1, 1, Sc, 1)


def _ballq(xyz_b, new_t, K, radius, Sc=128):
    B, three, N = xyz_b.shape
    S = new_t.shape[1]
    grid = (B, S // Sc)
    slot, cnt = pl.pallas_call(
        functools.partial(_ballq_body, N, K, radius * radius),
        grid=grid,
        in_specs=[
            pl.BlockSpec((1, 3, N), lambda b, s: (b, 0, 0)),
            pl.BlockSpec((1, Sc, 3), lambda b, s: (b, s, 0)),
        ],
        out_specs=[
            pl.BlockSpec((1, Sc, N), lambda b, s: (b, s, 0)),
            pl.BlockSpec((1, 1, Sc, 1), lambda b, s: (b, s, 0, 0)),
        ],
        out_shape=[
            jax.ShapeDtypeStruct((B, S, N), jnp.int32),
            jax.ShapeDtypeStruct((B, S // Sc, Sc, 1), jnp.int32),
        ],
    )(xyz_b, new_t)
    return slot, cnt


def _sc_group_gather(slot_flat, table, B, S, N, K, Dg):
    BS = B * S
    NW = 32
    rows_per_w = BS // NW
    s_shift = int(math.log2(S))
    mesh = plsc.VectorSubcoreMesh(core_axis_name="c", subcore_axis_name="s")

    @functools.partial(
        pl.kernel,
        mesh=mesh,
        compiler_params=pltpu.CompilerParams(
            needs_layout_passes=False, use_tc_tiling_on_sc=False),
        out_type=jax.ShapeDtypeStruct((BS * K, Dg), jnp.float32),
        scratch_types=[
            pltpu.VMEM((N,), jnp.int32),
            pltpu.VMEM((K,), jnp.int32),
            pltpu.VMEM((K, Dg), jnp.float32),
            pltpu.SemaphoreType.DMA,
        ],
    )
    def kern(slot_hbm, table_hbm, out_hbm, slot_v, gidx_v, rows_v, sem):
        wid = lax.axis_index("s") * 2 + lax.axis_index("c")
        lane = lax.iota(jnp.int32, 16)
        zero16 = jnp.zeros((16,), jnp.int32)

        def row_body(i, _):
            r = wid * rows_per_w + i
            b = lax.shift_right_logical(r, s_shift)
            base = b * N
            pltpu.sync_copy(slot_hbm.at[pl.ds(r * N, N)], slot_v)
            for kk in range(K // 16):
                gidx_v[pl.ds(kk * 16, 16)] = zero16

            def chunk(c, _):
                sl = slot_v[pl.ds(c * 16, 16)]
                pred = sl >= 0
                nidx = lane + (base + c * 16)
                plsc.store_scatter(gidx_v, [sl], nidx, mask=pred)
                return 0

            lax.fori_loop(0, N // 16, chunk, 0)
            pltpu.async_copy(table_hbm.at[gidx_v], rows_v, sem).wait()
            pltpu.sync_copy(rows_v, out_hbm.at[pl.ds(r * K, K)])
            return 0

        lax.fori_loop(0, rows_per_w, row_body, 0)

    return kern(slot_flat, table)


def _branch_body(K, Dg, preact, nlayers, *refs):
    g_ref, new_ref, cnt_ref = refs[0], refs[1], refs[2]
    wrefs = refs[3:-1]
    out_ref = refs[-1]
    Sc = new_ref.shape[1]
    g = g_ref[...].reshape(Sc * K, Dg)
    c3 = new_ref[...].reshape(Sc, 3)
    cx, cy, cz = c3[:, 0:1], c3[:, 1:2], c3[:, 2:3]

    wi = 0
    if not preact:
        Wg = wrefs[wi][...]
        wi += 1
        z = jnp.dot(g, Wg, preferred_element_type=jnp.float32)
    else:
        z = g
    Wc3 = wrefs[wi][...]
    b1 = wrefs[wi + 1][...]
    wi += 2
    off = b1 - ((cx * Wc3[0:1] + cy * Wc3[1:2]) + cz * Wc3[2:3])
    C1 = off.shape[1]
    z = z.reshape(Sc, K, C1) + off[:, None, :]
    h = jnp.maximum(z, 0.0).reshape(Sc * K, C1)
    for _ in range(nlayers - 1):
        W = wrefs[wi][...]
        b = wrefs[wi + 1][...]
        wi += 2
        h = jnp.maximum(jnp.dot(h, W, preferred_element_type=jnp.float32) + b, 0.0)
    Cout = h.shape[1]
    cnt = cnt_ref[...].reshape(Sc, 1)
    kio = lax.broadcasted_iota(jnp.int32, (Sc, K), 1)
    valid = jnp.where(kio < cnt, 1.0, 0.0)
    hv = h.reshape(Sc, K, Cout) * valid[:, :, None]
    out_ref[...] = jnp.max(hv, axis=1).reshape(1, Sc, Cout)


def _branch_mlp(G, new_t, cnt, folded, preact, Wg=None, Sc=None):
    B, S, K, Dg = G.shape
    Sc = Sc or max(8, 4096 // K)
    W1, b1 = folded[0]
    Wc3 = W1[-3:]
    ops = [G, new_t, cnt.reshape(B, S // Sc, Sc, 1)]
    wlist = []
    if not preact:
        wlist.append(Wg)
    wlist += [Wc3, b1[None, :]]
    for (W, b) in folded[1:]:
        wlist += [W, b[None, :]]
    Cout = folded[-1][0].shape[1]
    in_specs = [
        pl.BlockSpec((1, Sc, K, Dg), lambda b, s: (b, s, 0, 0)),
        pl.BlockSpec((1, Sc, 3), lambda b, s: (b, s, 0)),
        pl.BlockSpec((1, 1, Sc, 1), lambda b, s: (b, s, 0, 0)),
    ]
    for w in wlist:
        in_specs.append(
            pl.BlockSpec(w.shape, functools.partial(
                lambda nd, b, s: tuple(0 for _ in range(nd)), w.ndim)))
    return pl.pallas_call(
        functools.partial(_branch_body, K, Dg, preact, len(folded)),
        grid=(B, S // Sc),
        in_specs=in_specs,
        out_specs=pl.BlockSpec((1, Sc, Cout), lambda b, s: (b, s, 0)),
        out_shape=jax.ShapeDtypeStruct((B, S, Cout), jnp.float32),
    )(*ops, *wlist)


def _dense_body(x_ref, w_ref, o_ref):
    o_ref[...] = jnp.dot(x_ref[...], w_ref[...], preferred_element_type=jnp.float32)


def _dense(x, W, Mc=1024):
    M, Cin = x.shape
    Cout = W.shape[1]
    return pl.pallas_call(
        _dense_body,
        grid=(M // Mc,),
        in_specs=[
            pl.BlockSpec((Mc, Cin), lambda i: (i, 0)),
            pl.BlockSpec(W.shape, lambda i: (0, 0)),
        ],
        out_specs=pl.BlockSpec((Mc, Cout), lambda i: (i, 0)),
        out_shape=jax.ShapeDtypeStruct((M, Cout), jnp.float32),
    )(x, W)


def _sa3fp3_body(f_ref, w1, b1, w2, b2, w3, b3, wa, wb, bf1, w2f, b2f,
                 l3_ref, fp3_ref):
    f = f_ref[...].reshape(f_ref.shape[1], f_ref.shape[2])
    h = jnp.maximum(jnp.dot(f, w1[...], preferred_element_type=jnp.float32) + b1[...], 0.0)
    h = jnp.maximum(jnp.dot(h, w2[...], preferred_element_type=jnp.float32) + b2[...], 0.0)
    h = jnp.maximum(jnp.dot(h, w3[...], preferred_element_type=jnp.float32) + b3[...], 0.0)
    l3 = jnp.max(h, axis=0, keepdims=True)
    l3_ref[...] = l3.reshape(1, 1, l3.shape[1])
    l2p = f[:, 3:]
    g = jnp.dot(l2p, wa[...], preferred_element_type=jnp.float32)
    g = g + jnp.dot(l3, wb[...], preferred_element_type=jnp.float32)
    g = jnp.maximum(g + bf1[...], 0.0)
    g = jnp.maximum(jnp.dot(g, w2f[...], preferred_element_type=jnp.float32) + b2f[...], 0.0)
    fp3_ref[...] = g.reshape(1, g.shape[0], g.shape[1])


def _sa3fp3(f_in, sa3_f, fp3_f):
    B, S2, C = f_in.shape
    (W1, b1), (W2, b2), (W3, b3) = sa3_f
    (Wf1, bf1), (W2f, b2f) = fp3_f
    Wa, Wb = Wf1[:512], Wf1[512:]
    ws = [W1, b1[None], W2, b2[None], W3, b3[None],
          Wa, Wb, bf1[None], W2f, b2f[None]]
    in_specs = [pl.BlockSpec((1, S2, C), lambda b: (b, 0, 0))]
    for w in ws:
        in_specs.append(
            pl.BlockSpec(w.shape, functools.partial(
                lambda nd, b: tuple(0 for _ in range(nd)), w.ndim)))
    return pl.pallas_call(
        _sa3fp3_body,
        grid=(B,),
        in_specs=in_specs,
        out_specs=[
            pl.BlockSpec((1, 1, 1024), lambda b: (b, 0, 0)),
            pl.BlockSpec((1, S2, 256), lambda b: (b, 0, 0)),
        ],
        out_shape=[
            jax.ShapeDtypeStruct((B, 1, 1024), jnp.float32),
            jax.ShapeDtypeStruct((B, S2, 256), jnp.float32),
        ],
    )(f_in, *ws)


def _interp_weights(x1, x2x, x2y, x2z, S):
    cx, cy, cz = x1[:, 0:1], x1[:, 1:2], x1[:, 2:3]
    c2 = (cx * cx + cy * cy) + cz * cz
    x2n = (x2x * x2x + x2y * x2y) + x2z * x2z
    cross = (cx * x2x + cy * x2y) + cz * x2z
    d = (c2 + x2n) - 2.0 * cross
    Nc = d.shape[0]
    iota = lax.broadcasted_iota(jnp.int32, (Nc, S), 1)
    BIG = jnp.float32(3.0e38)
    ws, ohs = [], []
    dd = d
    for _ in range(3):
        m = jnp.min(dd, axis=1, keepdims=True)
        idx = jnp.min(jnp.where(dd == m, iota, S), axis=1, keepdims=True)
        oh = iota == idx
        ws.append(1.0 / (m + 1e-8))
        ohs.append(oh)
        dd = jnp.where(oh, BIG, dd)
    norm = (ws[0] + ws[1]) + ws[2]
    W = jnp.where(ohs[0], ws[0] / norm, 0.0)
    W = W + jnp.where(ohs[1], ws[1] / norm, 0.0)
    W = W + jnp.where(ohs[2], ws[2] / norm, 0.0)
    return W


def _fp2_body(S, x1_ref, x2_ref, p2_ref, p1_ref, wa, wb, b1, w2, b2, out_ref):
    Nc = x1_ref.shape[1]
    x1 = x1_ref[...].reshape(Nc, 3)
    x2 = x2_ref[...].reshape(3, S)
    Wm = _interp_weights(x1, x2[0:1], x2[1:2], x2[2:3], S)
    p2 = p2_ref[...].reshape(S, p2_ref.shape[2])
    interp = jnp.dot(Wm, p2, preferred_element_type=jnp.float32)
    p1 = p1_ref[...].reshape(Nc, p1_ref.shape[2])
    h = jnp.dot(p1, wa[...], preferred_element_type=jnp.float32)
    h = h + jnp.dot(interp, wb[...], preferred_element_type=jnp.float32)
    h = jnp.maximum(h + b1[...], 0.0)
    h = jnp.maximum(jnp.dot(h, w2[...], preferred_element_type=jnp.float32) + b2[...], 0.0)
    out_ref[...] = h.reshape(1, Nc, h.shape[1])


def _fp2(x1_t, x2_b, p2, p1, fp2_f):
    B, N1, _ = x1_t.shape
    S = x2_b.shape[2]
    C2 = p2.shape[2]
    Cp1 = p1.shape[2]
    (W1, b1), (W2, b2) = fp2_f
    Wa, Wb = W1[:Cp1], W1[Cp1:]
    ws = [Wa, Wb, b1[None], W2, b2[None]]
    in_specs = [
        pl.BlockSpec((1, N1, 3), lambda b: (b, 0, 0)),
        pl.BlockSpec((1, 3, S), lambda b: (b, 0, 0)),
        pl.BlockSpec((1, S, C2), lambda b: (b, 0, 0)),
        pl.BlockSpec((1, N1, Cp1), lambda b: (b, 0, 0)),
    ]
    for w in ws:
        in_specs.append(
            pl.BlockSpec(w.shape, functools.partial(
                lambda nd, b: tuple(0 for _ in range(nd)), w.ndim)))
    Cout = W2.shape[1]
    return pl.pallas_call(
        functools.partial(_fp2_body, S),
        grid=(B,),
        in_specs=in_specs,
        out_specs=pl.BlockSpec((1, N1, Cout), lambda b: (b, 0, 0)),
        out_shape=jax.ShapeDtypeStruct((B, N1, Cout), jnp.float32),
    )(x1_t, x2_b, p2, p1, *ws)


def _fp1_body(S, x1_ref, x2_ref, p2_ref, cls_ref, wxyz, w0, wb, b1, w2, b2,
              wh1, bh1, wh2, bh2, out_ref):
    Nc = x1_ref.shape[1]
    x1 = x1_ref[...].reshape(Nc, 3)
    x2 = x2_ref[...].reshape(3, S)
    Wm = _interp_weights(x1, x2[0:1], x2[1:2], x2[2:3], S)
    p2 = p2_ref[...].reshape(S, p2_ref.shape[2])
    interp = jnp.dot(Wm, p2, preferred_element_type=jnp.float32)
    cx, cy, cz = x1[:, 0:1], x1[:, 1:2], x1[:, 2:3]
    wx = wxyz[...]
    z = (cx * wx[0:1] + cy * wx[1:2]) + cz * wx[2:3]
    cls = cls_ref[0, 0, 0]
    z = z + cls * w0[...]
    z = z + jnp.dot(interp, wb[...], preferred_element_type=jnp.float32)
    h = jnp.maximum(z + b1[...], 0.0)
    h = jnp.maximum(jnp.dot(h, w2[...], preferred_element_type=jnp.float32) + b2[...], 0.0)
    h = jnp.maximum(jnp.dot(h, wh1[...], preferred_element_type=jnp.float32) + bh1[...], 0.0)
    o = jnp.dot(h, wh2[...], preferred_element_type=jnp.float32) + bh2[...]
    m = jnp.max(o, axis=1, keepdims=True)
    shifted = o - m
    lse = jnp.log(jnp.sum(jnp.exp(shifted), axis=1, keepdims=True))
    out_ref[...] = (shifted - lse).reshape(1, Nc, o.shape[1])


def _fp1_head(x1_t, x2_b, p2, cls_label, fp1_f, head1_f, head2, Nc=1024):
    B, N0, _ = x1_t.shape
    S = x2_b.shape[2]
    C2 = p2.shape[2]
    (W1, b1), (W2, b2) = fp1_f
    w0 = W1[0:1]
    Wxyz = W1[1:4] + W1[4:7]
    Wb = W1[7:]
    Wh1, bh1 = head1_f
    Wh2, bh2 = head2
    cls3 = cls_label.reshape(B, 1, 1)
    ws = [Wxyz, w0, Wb, b1[None], W2, b2[None], Wh1, bh1[None], Wh2, bh2[None]]
    in_specs = [
        pl.BlockSpec((1, Nc, 3), lambda b, i: (b, i, 0)),
        pl.BlockSpec((1, 3, S), lambda b, i: (b, 0, 0)),
        pl.BlockSpec((1, S, C2), lambda b, i: (b, 0, 0)),
        pl.BlockSpec((1, 1, 1), lambda b, i: (b, 0, 0)),
    ]
    for w in ws:
        in_specs.append(
            pl.BlockSpec(w.shape, functools.partial(
                lambda nd, b, i: tuple(0 for _ in range(nd)), w.ndim)))
    nclass = Wh2.shape[1]
    return pl.pallas_call(
        functools.partial(_fp1_body, S),
        grid=(B, N0 // Nc),
        in_specs=in_specs,
        out_specs=pl.BlockSpec((1, Nc, nclass), lambda b, i: (b, i, 0)),
        out_shape=jax.ShapeDtypeStruct((B, N0, nclass), jnp.float32),
    )(x1_t, x2_b, p2, cls3, *ws)


def kernel(xyz, cls_label, params):
    B, _, N0 = xyz.shape
    xyz0_sep = jnp.transpose(xyz, (1, 0, 2))
    xyz0_t = jnp.transpose(xyz, (0, 2, 1))

    S1 = 1024
    l1_sep = _fps(xyz0_sep, S1)
    l1_t = jnp.transpose(l1_sep, (1, 2, 0))
    l1_b = jnp.transpose(l1_sep, (1, 0, 2))

    table1 = jnp.concatenate(
        [xyz0_t.reshape(B * N0, 3),
         jnp.zeros((B * N0, 5), jnp.float32)], axis=1)

    sa1_cfg = [(0.1, 32), (0.2, 64), (0.4, 128)]
    l1_outs = []
    for (radius, K), layers in zip(sa1_cfg, params['sa1']):
        folded = _fold(layers)
        W1, b1 = folded[0]
        Wg = jnp.concatenate([W1[0:3] + W1[3:6],
                              jnp.zeros((5, W1.shape[1]), jnp.float32)], axis=0)
        slot, cnt = _ballq(xyz, l1_t, K, radius)
        G = _sc_group_gather(slot.reshape(-1), table1, B, S1, N0, K, 8)
        G = G.reshape(B, S1, K, 8)
        l1_outs.append(_branch_mlp(G, l1_t, cnt, folded, preact=False, Wg=Wg))
    l1_pts = jnp.concatenate(l1_outs, axis=-1)

    S2 = 512
    l2_sep = _fps(l1_sep, S2)
    l2_t = jnp.transpose(l2_sep, (1, 2, 0))
    l2_b = jnp.transpose(l2_sep, (1, 0, 2))

    feat2 = jnp.concatenate([l1_pts, l1_t], axis=-1).reshape(B * S1, 323)
    sa2_cfg = [(0.4, 64), (0.8, 128)]
    l2_outs = []
    for (radius, K), layers in zip(sa2_cfg, params['sa2']):
        folded = _fold(layers)
        W1, b1 = folded[0]
        A1 = _dense(feat2, W1)
        slot, cnt = _ballq(l1_b, l2_t, K, radius)
        G = _sc_group_gather(slot.reshape(-1), A1, B, S2, S1, K, W1.shape[1])
        G = G.reshape(B, S2, K, W1.shape[1])
        l2_outs.append(_branch_mlp(G, l2_t, cnt, folded, preact=True))
    l2_pts = jnp.concatenate(l2_outs, axis=-1)

    f_in = jnp.concatenate([l2_t, l2_pts], axis=-1)
    l3, fp3_out = _sa3fp3(f_in, _fold(params['sa3']), _fold(params['fp3']))

    fp2_out = _fp2(l1_t, l2_b, fp3_out, l1_pts, _fold(params['fp2']))

    W1h, b1h, g1h, be1h = params['head1']
    s = g1h / np.sqrt(1.0 + _BN_EPS)
    head1_f = (W1h * s[None, :], b1h * s + be1h)
    logp = _fp1_head(xyz0_t, l1_b, fp2_out, cls_label,
                     _fold(params['fp1']), head1_f, params['head2'])

    l3_points = jnp.transpose(l3, (0, 2, 1))
    return logp, l3_points


# speedup vs baseline: 29.1345x; 6.5004x over previous
import functools
import math

import jax
import jax.numpy as jnp
import numpy as np
from jax import lax
from jax.experimental import pallas as pl
from jax.experimental.pallas import tpu as pltpu
from jax.experimental.pallas import tpu_sc as plsc

_BN_EPS = 1e-5


def _fold(layers):
    out = []
    for (W, b, g, be) in layers:
        s = g / np.sqrt(1.0 + _BN_EPS)
        out.append((W * s[None, :], b * s + be))
    return out


def _fps_body(P, N, xyz_ref, out_ref):
    x = xyz_ref[0]
    y = xyz_ref[1]
    z = xyz_ref[2]
    B = x.shape[0]
    iota = lax.broadcasted_iota(jnp.int32, (B, N), 1)
    iotaP = lax.broadcasted_iota(jnp.int32, (B, P), 1)

    def body(i, state):
        dist, far, cxs, cys, czs = state
        oh = iota == far
        cx = jnp.sum(jnp.where(oh, x, 0.0), axis=1, keepdims=True)
        cy = jnp.sum(jnp.where(oh, y, 0.0), axis=1, keepdims=True)
        cz = jnp.sum(jnp.where(oh, z, 0.0), axis=1, keepdims=True)
        sel = iotaP == i
        cxs = jnp.where(sel, cx, cxs)
        cys = jnp.where(sel, cy, cys)
        czs = jnp.where(sel, cz, czs)
        dx = x - cx
        dy = y - cy
        dz = z - cz
        d = (dx * dx + dy * dy) + dz * dz
        dist = jnp.minimum(dist, d)
        m = jnp.max(dist, axis=1, keepdims=True)
        far = jnp.min(jnp.where(dist == m, iota, N), axis=1, keepdims=True)
        return dist, far, cxs, cys, czs

    zP = jnp.zeros((B, P), jnp.float32)
    _, _, cxs, cys, czs = lax.fori_loop(
        0, P,
        body,
        (jnp.full((B, N), 1e10, jnp.float32), jnp.zeros((B, 1), jnp.int32),
         zP, zP, zP),
    )
    out_ref[0] = cxs
    out_ref[1] = cys
    out_ref[2] = czs


def _fps(xyz_sep, P):
    three, B, N = xyz_sep.shape
    return pl.pallas_call(
        functools.partial(_fps_body, P, N),
        out_shape=jax.ShapeDtypeStruct((3, B, P), jnp.float32),
    )(xyz_sep)


def _ballq_body(N, K, r2, xyz_ref, new_ref, slot_ref, cnt_ref):
    xyz = xyz_ref[...].reshape(3, N)
    x = xyz[0:1]
    y = xyz[1:2]
    z = xyz[2:3]
    nw = new_ref[...]
    Sc = nw.shape[1]
    nw = nw.reshape(Sc, 3)
    cx = nw[:, 0:1]
    cy = nw[:, 1:2]
    cz = nw[:, 2:3]
    x2n = (x * x + y * y) + z * z
    c2 = (cx * cx + cy * cy) + cz * cz
    cross = (cx * x + cy * y) + cz * z
    sqr = (c2 + x2n) - 2.0 * cross
    mask = jnp.where(sqr <= r2, 1.0, 0.0).astype(jnp.float32)
    run = mask
    sh = 1
    while sh < N:
        shifted = jnp.concatenate(
            [jnp.zeros((Sc, sh), jnp.float32), run[:, : N - sh]], axis=1)
        run = run + shifted
        sh *= 2
    excl = run - mask
    sloti = excl.astype(jnp.int32)
    slot_ref[...] = jnp.where(
        (mask > 0.0) & (sloti < K), sloti, -1).reshape(1, Sc, N)
    cnt = jnp.minimum(run[:, N - 1 : N], float(K)).astype(jnp.int32)
    cnt_ref[...] = cnt.reshape(1, 1, Sc, 1)


def _ballq(xyz_b, new_t, K, radius, Sc=128):
    B, three, N = xyz_b.shape
    S = new_t.shape[1]
    grid = (B, S // Sc)
    slot, cnt = pl.pallas_call(
        functools.partial(_ballq_body, N, K, radius * radius),
        grid=grid,
        in_specs=[
            pl.BlockSpec((1, 3, N), lambda b, s: (b, 0, 0)),
            pl.BlockSpec((1, Sc, 3), lambda b, s: (b, s, 0)),
        ],
        out_specs=[
            pl.BlockSpec((1, Sc, N), lambda b, s: (b, s, 0)),
            pl.BlockSpec((1, 1, Sc, 1), lambda b, s: (b, s, 0, 0)),
        ],
        out_shape=[
            jax.ShapeDtypeStruct((B, S, N), jnp.int32),
            jax.ShapeDtypeStruct((B, S // Sc, Sc, 1), jnp.int32),
        ],
    )(xyz_b, new_t)
    return slot, cnt


def _sc_group_gather(slot_flat, table_flat, B, S, N, K, RW, dtype):
    BS = B * S
    NW = 32
    rows_per_w = BS // NW
    s_shift = int(math.log2(S))
    mesh = plsc.VectorSubcoreMesh(core_axis_name="c", subcore_axis_name="s")

    @functools.partial(
        pl.kernel,
        mesh=mesh,
        compiler_params=pltpu.CompilerParams(
            needs_layout_passes=False, use_tc_tiling_on_sc=False),
        out_type=jax.ShapeDtypeStruct((BS * K * RW,), dtype),
        scratch_types=[
            pltpu.VMEM((N * RW,), dtype),
            pltpu.VMEM((N,), jnp.int32),
            pltpu.VMEM((K,), jnp.int32),
            pltpu.VMEM((K * RW,), dtype),
        ],
    )
    def kern(slot_hbm, table_hbm, out_hbm, tab_v, slot_v, gidx_v, rows_v):
        wid = lax.axis_index("s") * 2 + lax.axis_index("c")
        b = lax.shift_right_logical(wid * rows_per_w, s_shift)
        pltpu.sync_copy(table_hbm.at[pl.ds(b * (N * RW), N * RW)], tab_v)
        lane = lax.iota(jnp.int32, 16)
        zero16 = jnp.zeros((16,), jnp.int32)

        def row_body(i, _):
            r = wid * rows_per_w + i
            pltpu.sync_copy(slot_hbm.at[pl.ds(r * N, N)], slot_v)
            for kk in range(K // 16):
                gidx_v[pl.ds(kk * 16, 16)] = zero16

            def chunk(c, _):
                sl = slot_v[pl.ds(c * 16, 16)]
                pred = sl >= 0
                nidx = lane + c * 16
                plsc.store_scatter(gidx_v, [sl], nidx, mask=pred)
                return 0

            lax.fori_loop(0, N // 16, chunk, 0)

            def gat16(kg, _):
                rows16 = gidx_v[pl.ds(kg * 16, 16)]
                for j in range(16):
                    row = rows16[j]
                    for jj in range(RW // 16):
                        rows_v[pl.ds((kg * 16 + j) * RW + jj * 16, 16)] = (
                            tab_v[pl.ds(row * RW + jj * 16, 16)])
                return 0

            lax.fori_loop(0, K // 16, gat16, 0)
            pltpu.sync_copy(rows_v, out_hbm.at[pl.ds(r * (K * RW), K * RW)])
            return 0

        lax.fori_loop(0, rows_per_w, row_body, 0)

    return kern(slot_flat, table_flat)


def _branch_body(K, Dg, preact, nlayers, *refs):
    g_ref, new_ref, cnt_ref = refs[0], refs[1], refs[2]
    wrefs = refs[3:-1]
    out_ref = refs[-1]
    Sc = new_ref.shape[1]
    g = g_ref[...].reshape(Sc * K, Dg)
    c3 = new_ref[...].reshape(Sc, 3)
    cx, cy, cz = c3[:, 0:1], c3[:, 1:2], c3[:, 2:3]

    wi = 0
    if not preact:
        Wg = wrefs[wi][...]
        wi += 1
        z = jnp.dot(g, Wg, preferred_element_type=jnp.float32)
    else:
        z = g.astype(jnp.float32)
    Wc3 = wrefs[wi][...]
    b1 = wrefs[wi + 1][...]
    wi += 2
    off = b1 - ((cx * Wc3[0:1] + cy * Wc3[1:2]) + cz * Wc3[2:3])
    C1 = off.shape[1]
    z = z.reshape(Sc, K, C1) + off[:, None, :]
    h = jnp.maximum(z, 0.0).reshape(Sc * K, C1)
    for _ in range(nlayers - 1):
        W = wrefs[wi][...]
        b = wrefs[wi + 1][...]
        wi += 2
        h = jnp.maximum(jnp.dot(h, W, preferred_element_type=jnp.float32) + b, 0.0)
    Cout = h.shape[1]
    cnt = cnt_ref[...].reshape(Sc, 1)
    kio = lax.broadcasted_iota(jnp.int32, (Sc, K), 1)
    valid = jnp.where(kio < cnt, 1.0, 0.0)
    hv = h.reshape(Sc, K, Cout) * valid[:, :, None]
    out_ref[...] = jnp.max(hv, axis=1).reshape(1, Sc, Cout)


def _branch_mlp(G, new_t, cnt, folded, preact, Wg=None, Sc=None):
    B, S, K, Dg = G.shape
    Sc = Sc or max(8, 4096 // K)
    W1, b1 = folded[0]
    Wc3 = W1[-3:]
    ops = [G, new_t, cnt.reshape(B, S // Sc, Sc, 1)]
    wlist = []
    if not preact:
        wlist.append(Wg)
    wlist += [Wc3, b1[None, :]]
    for (W, b) in folded[1:]:
        wlist += [W, b[None, :]]
    Cout = folded[-1][0].shape[1]
    in_specs = [
        pl.BlockSpec((1, Sc, K, Dg), lambda b, s: (b, s, 0, 0)),
        pl.BlockSpec((1, Sc, 3), lambda b, s: (b, s, 0)),
        pl.BlockSpec((1, 1, Sc, 1), lambda b, s: (b, s, 0, 0)),
    ]
    for w in wlist:
        in_specs.append(
            pl.BlockSpec(w.shape, functools.partial(
                lambda nd, b, s: tuple(0 for _ in range(nd)), w.ndim)))
    return pl.pallas_call(
        functools.partial(_branch_body, K, Dg, preact, len(folded)),
        grid=(B, S // Sc),
        in_specs=in_specs,
        out_specs=pl.BlockSpec((1, Sc, Cout), lambda b, s: (b, s, 0)),
        out_shape=jax.ShapeDtypeStruct((B, S, Cout), jnp.float32),
    )(*ops, *wlist)


def _dense_body(x_ref, w_ref, o_ref):
    o_ref[...] = jnp.dot(x_ref[...], w_ref[...], preferred_element_type=jnp.float32)


def _dense(x, W, Mc=1024):
    M, Cin = x.shape
    Cout = W.shape[1]
    return pl.pallas_call(
        _dense_body,
        grid=(M // Mc,),
        in_specs=[
            pl.BlockSpec((Mc, Cin), lambda i: (i, 0)),
            pl.BlockSpec(W.shape, lambda i: (0, 0)),
        ],
        out_specs=pl.BlockSpec((Mc, Cout), lambda i: (i, 0)),
        out_shape=jax.ShapeDtypeStruct((M, Cout), jnp.float32),
    )(x, W)


def _sa3fp3_body(f_ref, w1, b1, w2, b2, w3, b3, wa, wb, bf1, w2f, b2f,
                 l3_ref, fp3_ref):
    f = f_ref[...].reshape(f_ref.shape[1], f_ref.shape[2])
    h = jnp.maximum(jnp.dot(f, w1[...], preferred_element_type=jnp.float32) + b1[...], 0.0)
    h = jnp.maximum(jnp.dot(h, w2[...], preferred_element_type=jnp.float32) + b2[...], 0.0)
    h = jnp.maximum(jnp.dot(h, w3[...], preferred_element_type=jnp.float32) + b3[...], 0.0)
    l3 = jnp.max(h, axis=0, keepdims=True)
    l3_ref[...] = l3.reshape(1, 1, l3.shape[1])
    l2p = f[:, 3:]
    g = jnp.dot(l2p, wa[...], preferred_element_type=jnp.float32)
    g = g + jnp.dot(l3, wb[...], preferred_element_type=jnp.float32)
    g = jnp.maximum(g + bf1[...], 0.0)
    g = jnp.maximum(jnp.dot(g, w2f[...], preferred_element_type=jnp.float32) + b2f[...], 0.0)
    fp3_ref[...] = g.reshape(1, g.shape[0], g.shape[1])


def _sa3fp3(f_in, sa3_f, fp3_f):
    B, S2, C = f_in.shape
    (W1, b1), (W2, b2), (W3, b3) = sa3_f
    (Wf1, bf1), (W2f, b2f) = fp3_f
    Wa, Wb = Wf1[:512], Wf1[512:]
    ws = [W1, b1[None], W2, b2[None], W3, b3[None],
          Wa, Wb, bf1[None], W2f, b2f[None]]
    in_specs = [pl.BlockSpec((1, S2, C), lambda b: (b, 0, 0))]
    for w in ws:
        in_specs.append(
            pl.BlockSpec(w.shape, functools.partial(
                lambda nd, b: tuple(0 for _ in range(nd)), w.ndim)))
    return pl.pallas_call(
        _sa3fp3_body,
        grid=(B,),
        in_specs=in_specs,
        out_specs=[
            pl.BlockSpec((1, 1, 1024), lambda b: (b, 0, 0)),
            pl.BlockSpec((1, S2, 256), lambda b: (b, 0, 0)),
        ],
        out_shape=[
            jax.ShapeDtypeStruct((B, 1, 1024), jnp.float32),
            jax.ShapeDtypeStruct((B, S2, 256), jnp.float32),
        ],
    )(f_in, *ws)


def _interp_weights(x1, x2x, x2y, x2z, S):
    cx, cy, cz = x1[:, 0:1], x1[:, 1:2], x1[:, 2:3]
    c2 = (cx * cx + cy * cy) + cz * cz
    x2n = (x2x * x2x + x2y * x2y) + x2z * x2z
    cross = (cx * x2x + cy * x2y) + cz * x2z
    d = (c2 + x2n) - 2.0 * cross
    Nc = d.shape[0]
    iota = lax.broadcasted_iota(jnp.int32, (Nc, S), 1)
    BIG = jnp.float32(3.0e38)
    ws, ohs = [], []
    dd = d
    for _ in range(3):
        m = jnp.min(dd, axis=1, keepdims=True)
        idx = jnp.min(jnp.where(dd == m, iota, S), axis=1, keepdims=True)
        oh = iota == idx
        ws.append(1.0 / (m + 1e-8))
        ohs.append(oh)
        dd = jnp.where(oh, BIG, dd)
    norm = (ws[0] + ws[1]) + ws[2]
    W = jnp.where(ohs[0], ws[0] / norm, 0.0)
    W = W + jnp.where(ohs[1], ws[1] / norm, 0.0)
    W = W + jnp.where(ohs[2], ws[2] / norm, 0.0)
    return W


def _fp2_body(S, x1_ref, x2_ref, p2_ref, p1_ref, wa, wb, b1, w2, b2, out_ref):
    Nc = x1_ref.shape[1]
    x1 = x1_ref[...].reshape(Nc, 3)
    x2 = x2_ref[...].reshape(3, S)
    Wm = _interp_weights(x1, x2[0:1], x2[1:2], x2[2:3], S)
    p2 = p2_ref[...].reshape(S, p2_ref.shape[2])
    interp = jnp.dot(Wm, p2, preferred_element_type=jnp.float32)
    p1 = p1_ref[...].reshape(Nc, p1_ref.shape[2])
    h = jnp.dot(p1, wa[...], preferred_element_type=jnp.float32)
    h = h + jnp.dot(interp, wb[...], preferred_element_type=jnp.float32)
    h = jnp.maximum(h + b1[...], 0.0)
    h = jnp.maximum(jnp.dot(h, w2[...], preferred_element_type=jnp.float32) + b2[...], 0.0)
    out_ref[...] = h.reshape(1, Nc, h.shape[1])


def _fp2(x1_t, x2_b, p2, p1, fp2_f):
    B, N1, _ = x1_t.shape
    S = x2_b.shape[2]
    C2 = p2.shape[2]
    Cp1 = p1.shape[2]
    (W1, b1), (W2, b2) = fp2_f
    Wa, Wb = W1[:Cp1], W1[Cp1:]
    ws = [Wa, Wb, b1[None], W2, b2[None]]
    in_specs = [
        pl.BlockSpec((1, N1, 3), lambda b: (b, 0, 0)),
        pl.BlockSpec((1, 3, S), lambda b: (b, 0, 0)),
        pl.BlockSpec((1, S, C2), lambda b: (b, 0, 0)),
        pl.BlockSpec((1, N1, Cp1), lambda b: (b, 0, 0)),
    ]
    for w in ws:
        in_specs.append(
            pl.BlockSpec(w.shape, functools.partial(
                lambda nd, b: tuple(0 for _ in range(nd)), w.ndim)))
    Cout = W2.shape[1]
    return pl.pallas_call(
        functools.partial(_fp2_body, S),
        grid=(B,),
        in_specs=in_specs,
        out_specs=pl.BlockSpec((1, N1, Cout), lambda b: (b, 0, 0)),
        out_shape=jax.ShapeDtypeStruct((B, N1, Cout), jnp.float32),
    )(x1_t, x2_b, p2, p1, *ws)


def _fp1_body(S, x1_ref, x2_ref, p2_ref, cls_ref, wxyz, w0, wb, b1, w2, b2,
              wh1, bh1, wh2, bh2, out_ref):
    Nc = x1_ref.shape[1]
    x1 = x1_ref[...].reshape(Nc, 3)
    x2 = x2_ref[...].reshape(3, S)
    Wm = _interp_weights(x1, x2[0:1], x2[1:2], x2[2:3], S)
    p2 = p2_ref[...].reshape(S, p2_ref.shape[2])
    interp = jnp.dot(Wm, p2, preferred_element_type=jnp.float32)
    cx, cy, cz = x1[:, 0:1], x1[:, 1:2], x1[:, 2:3]
    wx = wxyz[...]
    z = (cx * wx[0:1] + cy * wx[1:2]) + cz * wx[2:3]
    cls = cls_ref[0, 0, 0]
    z = z + cls * w0[...]
    z = z + jnp.dot(interp, wb[...], preferred_element_type=jnp.float32)
    h = jnp.maximum(z + b1[...], 0.0)
    h = jnp.maximum(jnp.dot(h, w2[...], preferred_element_type=jnp.float32) + b2[...], 0.0)
    h = jnp.maximum(jnp.dot(h, wh1[...], preferred_element_type=jnp.float32) + bh1[...], 0.0)
    o = jnp.dot(h, wh2[...], preferred_element_type=jnp.float32) + bh2[...]
    m = jnp.max(o, axis=1, keepdims=True)
    shifted = o - m
    lse = jnp.log(jnp.sum(jnp.exp(shifted), axis=1, keepdims=True))
    out_ref[...] = (shifted - lse).reshape(1, Nc, o.shape[1])


def _fp1_head(x1_t, x2_b, p2, cls_label, fp1_f, head1_f, head2, Nc=1024):
    B, N0, _ = x1_t.shape
    S = x2_b.shape[2]
    C2 = p2.shape[2]
    (W1, b1), (W2, b2) = fp1_f
    w0 = W1[0:1]
    Wxyz = W1[1:4] + W1[4:7]
    Wb = W1[7:]
    Wh1, bh1 = head1_f
    Wh2, bh2 = head2
    cls3 = cls_label.reshape(B, 1, 1)
    ws = [Wxyz, w0, Wb, b1[None], W2, b2[None], Wh1, bh1[None], Wh2, bh2[None]]
    in_specs = [
        pl.BlockSpec((1, Nc, 3), lambda b, i: (b, i, 0)),
        pl.BlockSpec((1, 3, S), lambda b, i: (b, 0, 0)),
        pl.BlockSpec((1, S, C2), lambda b, i: (b, 0, 0)),
        pl.BlockSpec((1, 1, 1), lambda b, i: (b, 0, 0)),
    ]
    for w in ws:
        in_specs.append(
            pl.BlockSpec(w.shape, functools.partial(
                lambda nd, b, i: tuple(0 for _ in range(nd)), w.ndim)))
    nclass = Wh2.shape[1]
    return pl.pallas_call(
        functools.partial(_fp1_body, S),
        grid=(B, N0 // Nc),
        in_specs=in_specs,
        out_specs=pl.BlockSpec((1, Nc, nclass), lambda b, i: (b, i, 0)),
        out_shape=jax.ShapeDtypeStruct((B, N0, nclass), jnp.float32),
    )(x1_t, x2_b, p2, cls3, *ws)


def kernel(xyz, cls_label, params):
    B, _, N0 = xyz.shape
    xyz0_sep = jnp.transpose(xyz, (1, 0, 2))
    xyz0_t = jnp.transpose(xyz, (0, 2, 1))

    S1 = 1024
    l1_sep = _fps(xyz0_sep, S1)
    l1_t = jnp.transpose(l1_sep, (1, 2, 0))
    l1_b = jnp.transpose(l1_sep, (1, 0, 2))

    table1 = jnp.concatenate(
        [xyz0_t.reshape(B * N0, 3),
         jnp.zeros((B * N0, 13), jnp.float32)], axis=1).reshape(-1)

    sa1_cfg = [(0.1, 32), (0.2, 64), (0.4, 128)]
    l1_outs = []
    for (radius, K), layers in zip(sa1_cfg, params['sa1']):
        folded = _fold(layers)
        W1, b1 = folded[0]
        Wg = jnp.concatenate([W1[0:3] + W1[3:6],
                              jnp.zeros((13, W1.shape[1]), jnp.float32)], axis=0)
        slot, cnt = _ballq(xyz, l1_t, K, radius)
        G = _sc_group_gather(slot.reshape(-1), table1, B, S1, N0, K, 16,
                             jnp.float32)
        G = G.reshape(B, S1, K, 16)
        l1_outs.append(_branch_mlp(G, l1_t, cnt, folded, preact=False, Wg=Wg))
    l1_pts = jnp.concatenate(l1_outs, axis=-1)

    S2 = 512
    l2_sep = _fps(l1_sep, S2)
    l2_t = jnp.transpose(l2_sep, (1, 2, 0))
    l2_b = jnp.transpose(l2_sep, (1, 0, 2))

    feat2 = jnp.concatenate([l1_pts, l1_t], axis=-1).reshape(B * S1, 323)
    sa2_cfg = [(0.4, 64), (0.8, 128)]
    l2_outs = []
    for (radius, K), layers in zip(sa2_cfg, params['sa2']):
        folded = _fold(layers)
        W1, b1 = folded[0]
        C1 = W1.shape[1]
        A1 = _dense(feat2, W1)
        A1p = lax.bitcast_convert_type(
            A1.astype(jnp.bfloat16).reshape(B * S1, C1 // 2, 2),
            jnp.int32).reshape(-1)
        slot, cnt = _ballq(l1_b, l2_t, K, radius)
        Gw = _sc_group_gather(slot.reshape(-1), A1p, B, S2, S1, K, C1 // 2,
                              jnp.int32)
        G = lax.bitcast_convert_type(
            Gw.reshape(B, S2, K, C1 // 2), jnp.bfloat16).reshape(B, S2, K, C1)
        l2_outs.append(_branch_mlp(G, l2_t, cnt, folded, preact=True))
    l2_pts = jnp.concatenate(l2_outs, axis=-1)

    f_in = jnp.concatenate([l2_t, l2_pts], axis=-1)
    l3, fp3_out = _sa3fp3(f_in, _fold(params['sa3']), _fold(params['fp3']))

    fp2_out = _fp2(l1_t, l2_b, fp3_out, l1_pts, _fold(params['fp2']))

    W1h, b1h, g1h, be1h = params['head1']
    s = g1h / np.sqrt(1.0 + _BN_EPS)
    head1_f = (W1h * s[None, :], b1h * s + be1h)
    logp = _fp1_head(xyz0_t, l1_b, fp2_out, cls_label,
                     _fold(params['fp1']), head1_f, params['head2'])

    l3_points = jnp.transpose(l3, (0, 2, 1))
    return logp, l3_points
